# Initial kernel scaffold; baseline (speedup 1.0000x reference)
import jax
import jax.numpy as jnp
from jax.experimental import pallas as pl


def kernel(x, W1, g1, b1, W2, g2, b2, W3, g3, b3, W4, g4, b4, W5, g5, b5, W6, g6, b6, W7, g7, b7, device):
    raise NotImplementedError("write your pallas kernel here")


# speedup vs baseline: 9.6282x; 9.6282x over previous
import dataclasses
import functools

import jax
import jax.numpy as jnp
import numpy as np
from jax import lax
from jax.experimental import pallas as pl
from jax.experimental.pallas import tpu as pltpu
from jax.experimental.pallas import tpu_sc as plsc

_K = 20
_EPS = 1e-5


def _pairmat(C, groups):
  g = np.arange(C) // (C // groups)
  return jnp.asarray((g[:, None] == g[None, :]).astype(np.float32))


def _row(v):
  return v.reshape(1, -1)


def _dot(a, b):
  return jnp.dot(a, b, preferred_element_type=jnp.float32,
                 precision=lax.Precision.HIGHEST)


_BF = jnp.bfloat16


def _dotb(a, b):
  return jnp.dot(a.astype(_BF), b.astype(_BF),
                 preferred_element_type=jnp.float32)


def _knn_kernel(xt_ref, idx_ref, *, TR, k, N, C):
  r = pl.program_id(1)
  xall = xt_ref[0]
  rows = xt_ref[0, pl.ds(r * TR, TR), :]
  d = lax.dot_general(rows.astype(_BF), xall.astype(_BF),
                      (((1,), (1,)), ((), ())),
                      preferred_element_type=jnp.float32)
  inner = -2.0 * d
  ones = jnp.ones((1, C), jnp.float32)
  xx = lax.dot_general(ones, xall * xall, (((1,), (1,)), ((), ())),
                       preferred_element_type=jnp.float32,
                       precision=lax.Precision.HIGHEST)
  xxc = jnp.sum(rows * rows, axis=1, keepdims=True)
  s = (-xx - inner) - xxc
  colid = lax.broadcasted_iota(jnp.int32, (TR, N), 1)
  cols = []
  for t in range(k):
    m = jnp.max(s, axis=1, keepdims=True)
    cand = jnp.where(s >= m, colid, N)
    a = jnp.min(cand, axis=1, keepdims=True)
    cols.append(a)
    if t < k - 1:
      s = jnp.where(colid == a, -jnp.inf, s)
  idx_ref[0] = jnp.concatenate(cols, axis=1)


def _knn_topk(xt, k):
  B, N, C = xt.shape
  TR = 256
  return pl.pallas_call(
      functools.partial(_knn_kernel, TR=TR, k=k, N=N, C=C),
      grid=(B, N // TR),
      in_specs=[pl.BlockSpec((1, N, C), lambda b, r: (b, 0, 0))],
      out_specs=pl.BlockSpec((1, TR, k), lambda b, r: (b, r, 0)),
      out_shape=jax.ShapeDtypeStruct((B, N, k), jnp.int32),
  )(xt)


def _sc_gather(src, idx):
  M = idx.shape[1]
  C = src.shape[1]
  GW = 128
  mesh = plsc.VectorSubcoreMesh(core_axis_name="core",
                                subcore_axis_name="subcore")
  cp = pltpu.CompilerParams()
  if "needs_layout_passes" in type(cp).__dataclass_fields__:
    cp = dataclasses.replace(cp, needs_layout_passes=False)

  @functools.partial(
      pl.kernel,
      out_type=jax.ShapeDtypeStruct((M, C), src.dtype),
      mesh=mesh,
      compiler_params=cp,
  )
  def kern(x_hbm, i_hbm, o_hbm):
    def body(i_vmem, o_vmem):
      pltpu.sync_copy(x_hbm.at[i_vmem.at[0]], o_vmem)

    pltpu.emit_pipeline(
        body,
        grid=(M // GW,),
        in_specs=[pl.BlockSpec((1, GW), index_map=lambda i: (0, i))],
        out_specs=[pl.BlockSpec((GW, C), index_map=lambda i: (i, 0))],
        core_axis_name=("core", "subcore"),
        dimension_semantics=(pltpu.PARALLEL,),
    )(i_hbm, o_hbm)

  return kern(src, idx)


def _lrelu(h):
  return jnp.where(h >= 0, h, 0.2 * h)


def _gn_scale_shift(s1, s2, P, gamma, beta, cnt):
  mean = _dot(s1, P) / cnt
  e2 = _dot(s2, P) / cnt
  var = e2 - mean * mean
  inv = lax.rsqrt(var + _EPS)
  sc = gamma * inv
  return sc, beta - mean * sc


def _edge_kernel(two, xg_hbm, xt_ref, WeT_ref, ga_ref, ba_ref,
                 P_ref, *rest, K, N, Cg):
  if two:
    W2T_ref, g2_ref, b2_ref, out_ref, zs, buf, sem = rest
  else:
    out_ref, zs, buf, sem = rest
  b = pl.program_id(0)
  xc = xt_ref[0]
  WeT = WeT_ref[...]
  z0 = jnp.zeros((1, 128), jnp.float32)
  JJ = K // 2
  cnt = 2.0 * N * K

  def p1(jj, carry):
    s1, s2 = carry
    cp = pltpu.make_async_copy(xg_hbm.at[b, pl.ds(2 * jj, 2)], buf, sem)
    cp.start()
    cp.wait()
    ea = jnp.concatenate([buf[0, :, :Cg] - xc, xc], axis=1)
    eb = jnp.concatenate([buf[1, :, :Cg] - xc, xc], axis=1)
    za = _dotb(ea, WeT)
    zb = _dotb(eb, WeT)
    z = jnp.concatenate([za, zb], axis=1)
    zs[pl.ds(jj * N, N), :] = z
    return s1 + jnp.sum(z, 0, keepdims=True), s2 + jnp.sum(z * z, 0,
                                                           keepdims=True)

  s1, s2 = lax.fori_loop(0, JJ, p1, (z0, z0))
  sc, sh = _gn_scale_shift(s1, s2, P_ref[...], ga_ref[...], ba_ref[...], cnt)

  if two:
    W2T = W2T_ref[...]

    def p2(jj, carry):
      s1, s2 = carry
      h = _lrelu(zs[pl.ds(jj * N, N), :] * sc + sh)
      ya = _dotb(h[:, :64], W2T)
      yb = _dotb(h[:, 64:], W2T)
      y = jnp.concatenate([ya, yb], axis=1)
      zs[pl.ds(jj * N, N), :] = y
      return s1 + jnp.sum(y, 0, keepdims=True), s2 + jnp.sum(y * y, 0,
                                                             keepdims=True)

    s1, s2 = lax.fori_loop(0, JJ, p2, (z0, z0))
    sc, sh = _gn_scale_shift(s1, s2, P_ref[...], g2_ref[...], b2_ref[...],
                             cnt)

  def p3(jj, acc):
    h = _lrelu(zs[pl.ds(jj * N, N), :] * sc + sh)
    return jnp.maximum(acc, jnp.maximum(h[:, :64], h[:, 64:]))

  out_ref[0] = lax.fori_loop(0, JJ, p3,
                             jnp.full((N, 64), -jnp.inf, jnp.float32))


def _pairmat2(groups):
  g = np.arange(128) % 64 // (64 // groups)
  return jnp.asarray((g[:, None] == g[None, :]).astype(np.float32))


def _dup(v):
  return jnp.concatenate([v, v]).reshape(1, 128)


def _edge_block(xg, xt, WeT, ga, ba, W2T=None, g2=None, b2=None):
  B, K, N, Cw = xg.shape
  Cg = xt.shape[2]
  two = W2T is not None
  P = _pairmat2(32)
  ops = [xg, xt, WeT, _dup(ga), _dup(ba), P]
  in_specs = [
      pl.BlockSpec(memory_space=pl.ANY),
      pl.BlockSpec((1, N, Cg), lambda b: (b, 0, 0)),
      pl.BlockSpec(WeT.shape, lambda b: (0, 0)),
      pl.BlockSpec((1, 128), lambda b: (0, 0)),
      pl.BlockSpec((1, 128), lambda b: (0, 0)),
      pl.BlockSpec(P.shape, lambda b: (0, 0)),
  ]
  if two:
    ops += [W2T, _dup(g2), _dup(b2)]
    in_specs += [
        pl.BlockSpec(W2T.shape, lambda b: (0, 0)),
        pl.BlockSpec((1, 128), lambda b: (0, 0)),
        pl.BlockSpec((1, 128), lambda b: (0, 0)),
    ]
  return pl.pallas_call(
      functools.partial(_edge_kernel, two, K=K, N=N, Cg=Cg),
      grid=(B,),
      in_specs=in_specs,
      out_specs=pl.BlockSpec((1, N, 64), lambda b: (b, 0, 0)),
      out_shape=jax.ShapeDtypeStruct((B, N, 64), jnp.float32),
      scratch_shapes=[
          pltpu.VMEM((K // 2 * N, 128), jnp.float32),
          pltpu.VMEM((2, N, Cw), jnp.float32),
          pltpu.SemaphoreType.DMA,
      ],
  )(*ops)


def _head6_kernel(xc_ref, W6T_ref, g6_ref, b6_ref, P6_ref, out_ref, zs,
                  *, N, TN):
  nt = N // TN
  z0 = jnp.zeros((1, 1024), jnp.float32)
  W6T = W6T_ref[...]

  def p1(t, carry):
    s1, s2 = carry
    y = _dotb(xc_ref[0, pl.ds(t * TN, TN), :], W6T)
    zs[pl.ds(t * TN, TN), :] = y
    return s1 + jnp.sum(y, 0, keepdims=True), s2 + jnp.sum(y * y, 0,
                                                           keepdims=True)

  s1, s2 = lax.fori_loop(0, nt, p1, (z0, z0))
  sc, sh = _gn_scale_shift(s1, s2, P6_ref[...], g6_ref[...], b6_ref[...],
                           32.0 * N)

  def p2(t, acc):
    h = _lrelu(zs[pl.ds(t * TN, TN), :] * sc + sh)
    return jnp.maximum(acc, jnp.max(h, axis=0, keepdims=True))

  out_ref[0] = lax.fori_loop(0, nt, p2,
                             jnp.full((1, 1024), -jnp.inf, jnp.float32))


def _head6(xcat, W6T, g6, b6):
  B, N, _ = xcat.shape
  P6 = _pairmat(1024, 32)
  return pl.pallas_call(
      functools.partial(_head6_kernel, N=N, TN=512),
      grid=(B,),
      in_specs=[
          pl.BlockSpec((1, N, xcat.shape[2]), lambda b: (b, 0, 0)),
          pl.BlockSpec(W6T.shape, lambda b: (0, 0)),
          pl.BlockSpec((1, 1024), lambda b: (0, 0)),
          pl.BlockSpec((1, 1024), lambda b: (0, 0)),
          pl.BlockSpec(P6.shape, lambda b: (0, 0)),
      ],
      out_specs=pl.BlockSpec((1, 1, 1024), lambda b: (b, 0, 0)),
      out_shape=jax.ShapeDtypeStruct((B, 1, 1024), jnp.float32),
      scratch_shapes=[pltpu.VMEM((N, 1024), jnp.float32)],
  )(xcat, W6T, _row(g6), _row(b6), P6)


def _head7_kernel(xc_ref, fm_ref, WaT_ref, WbT_ref, g7_ref, b7_ref, P7_ref,
                  out_ref, zs, *, N, TN):
  nt = N // TN
  z0 = jnp.zeros((1, 512), jnp.float32)
  c7 = _dotb(fm_ref[0], WaT_ref[...])
  WbT = WbT_ref[...]

  def p1(t, carry):
    s1, s2 = carry
    y = _dotb(xc_ref[0, pl.ds(t * TN, TN), :], WbT) + c7
    zs[pl.ds(t * TN, TN), :] = y
    return s1 + jnp.sum(y, 0, keepdims=True), s2 + jnp.sum(y * y, 0,
                                                           keepdims=True)

  s1, s2 = lax.fori_loop(0, nt, p1, (z0, z0))
  sc, sh = _gn_scale_shift(s1, s2, P7_ref[...], g7_ref[...], b7_ref[...],
                           16.0 * N)

  def p2(t, _):
    out_ref[0, pl.ds(t * TN, TN), :] = _lrelu(
        zs[pl.ds(t * TN, TN), :] * sc + sh)
    return 0

  lax.fori_loop(0, nt, p2, 0)


def _head7(xcat, fmax, WaT, WbT, g7, b7):
  B, N, _ = xcat.shape
  P7 = _pairmat(512, 32)
  return pl.pallas_call(
      functools.partial(_head7_kernel, N=N, TN=512),
      grid=(B,),
      in_specs=[
          pl.BlockSpec((1, N, xcat.shape[2]), lambda b: (b, 0, 0)),
          pl.BlockSpec((1, 1, 1024), lambda b: (b, 0, 0)),
          pl.BlockSpec(WaT.shape, lambda b: (0, 0)),
          pl.BlockSpec(WbT.shape, lambda b: (0, 0)),
          pl.BlockSpec((1, 512), lambda b: (0, 0)),
          pl.BlockSpec((1, 512), lambda b: (0, 0)),
          pl.BlockSpec(P7.shape, lambda b: (0, 0)),
      ],
      out_specs=pl.BlockSpec((1, N, 512), lambda b: (b, 0, 0)),
      out_shape=jax.ShapeDtypeStruct((B, N, 512), jnp.float32),
      scratch_shapes=[pltpu.VMEM((N, 512), jnp.float32)],
  )(xcat, fmax, WaT, WbT, _row(g7), _row(b7), P7)


def _flat_idx(idx, B, N):
  base = (jnp.arange(B, dtype=jnp.int32) * N)[:, None, None]
  return (jnp.transpose(idx, (0, 2, 1)) + base).reshape(1, -1)


@jax.jit
def _run(x, W1, g1, b1, W2, g2, b2, W3, g3, b3, W4, g4, b4, W5, g5, b5,
         W6, g6, b6, W7, g7, b7):
  B, _, N = x.shape
  xt6 = jnp.transpose(x, (0, 2, 1))
  xt3p = jnp.pad(xt6[:, :, :3], ((0, 0), (0, 0), (0, 5)))
  xt6p = jnp.pad(xt6, ((0, 0), (0, 0), (0, 10)))

  idx1 = _knn_topk(xt3p, _K)
  src1 = jnp.pad(xt6p, ((0, 0), (0, 0), (0, 112))).reshape(B * N, 128)
  xg1 = _sc_gather(src1, _flat_idx(idx1, B, N))
  WeT1 = jnp.concatenate([jnp.pad(W1[:, :6].T, ((0, 10), (0, 0))),
                          jnp.pad(W1[:, 6:].T, ((0, 10), (0, 0)))], axis=0)
  x1 = _edge_block(xg1.reshape(B, _K, N, 128), xt6p, WeT1, g1, b1,
                   W2.T, g2, b2)

  idx2 = _knn_topk(x1, _K)
  src2 = jnp.pad(x1, ((0, 0), (0, 0), (0, 64))).reshape(B * N, 128)
  xg2 = _sc_gather(src2, _flat_idx(idx2, B, N))
  x2 = _edge_block(xg2.reshape(B, _K, N, 128), x1, W3.T, g3, b3,
                   W4.T, g4, b4)

  idx3 = _knn_topk(x2, _K)
  src3 = jnp.pad(x2, ((0, 0), (0, 0), (0, 64))).reshape(B * N, 128)
  xg3 = _sc_gather(src3, _flat_idx(idx3, B, N))
  x3 = _edge_block(xg3.reshape(B, _K, N, 128), x2, W5.T, g5, b5)

  xcat = jnp.concatenate([x1, x2, x3], axis=2)
  fmax = _head6(xcat, W6.T, g6, b6)
  out = _head7(xcat, fmax, W7[:, :1024].T, W7[:, 1024:].T, g7, b7)
  return jnp.transpose(out, (0, 2, 1))


def kernel(x, W1, g1, b1, W2, g2, b2, W3, g3, b3, W4, g4, b4, W5, g5, b5,
           W6, g6, b6, W7, g7, b7, device=0):
  return _run(x, W1, g1, b1, W2, g2, b2, W3, g3, b3, W4, g4, b4,
              W5, g5, b5, W6, g6, b6, W7, g7, b7)


# speedup vs baseline: 9.6444x; 1.0017x over previous
import dataclasses
import functools

import jax
import jax.numpy as jnp
import numpy as np
from jax import lax
from jax.experimental import pallas as pl
from jax.experimental.pallas import tpu as pltpu
from jax.experimental.pallas import tpu_sc as plsc

_K = 20
_EPS = 1e-5


def _pairmat(C, groups):
  g = np.arange(C) // (C // groups)
  return jnp.asarray((g[:, None] == g[None, :]).astype(np.float32))


def _row(v):
  return v.reshape(1, -1)


def _dot(a, b):
  return jnp.dot(a, b, preferred_element_type=jnp.float32,
                 precision=lax.Precision.HIGHEST)


_BF = jnp.bfloat16


def _dotb(a, b):
  return jnp.dot(a.astype(_BF), b.astype(_BF),
                 preferred_element_type=jnp.float32)


def _knn_kernel(xt_ref, idx_ref, *, TR, k, N, C):
  r = pl.program_id(1)
  xall = xt_ref[0]
  rows = xt_ref[0, pl.ds(r * TR, TR), :]
  d = lax.dot_general(rows.astype(_BF), xall.astype(_BF),
                      (((1,), (1,)), ((), ())),
                      preferred_element_type=jnp.float32)
  inner = -2.0 * d
  ones = jnp.ones((1, C), jnp.float32)
  xx = lax.dot_general(ones, xall * xall, (((1,), (1,)), ((), ())),
                       preferred_element_type=jnp.float32,
                       precision=lax.Precision.HIGHEST)
  xxc = jnp.sum(rows * rows, axis=1, keepdims=True)
  s = (-xx - inner) - xxc
  colid = lax.broadcasted_iota(jnp.int32, (TR, N), 1)
  cols = []
  for t in range(k):
    m = jnp.max(s, axis=1, keepdims=True)
    cand = jnp.where(s >= m, colid, N)
    a = jnp.min(cand, axis=1, keepdims=True)
    cols.append(a)
    if t < k - 1:
      s = jnp.where(colid == a, -jnp.inf, s)
  idx_ref[0] = jnp.concatenate(cols, axis=1)


def _knn_topk(xt, k):
  B, N, C = xt.shape
  TR = 256
  return pl.pallas_call(
      functools.partial(_knn_kernel, TR=TR, k=k, N=N, C=C),
      grid=(B, N // TR),
      in_specs=[pl.BlockSpec((1, N, C), lambda b, r: (b, 0, 0))],
      out_specs=pl.BlockSpec((1, TR, k), lambda b, r: (b, r, 0)),
      out_shape=jax.ShapeDtypeStruct((B, N, k), jnp.int32),
      compiler_params=pltpu.CompilerParams(
          dimension_semantics=("parallel", "arbitrary")),
  )(xt)


def _sc_gather(src, idx):
  M = idx.shape[1]
  C = src.shape[1]
  GW = 128
  mesh = plsc.VectorSubcoreMesh(core_axis_name="core",
                                subcore_axis_name="subcore")
  cp = pltpu.CompilerParams()
  if "needs_layout_passes" in type(cp).__dataclass_fields__:
    cp = dataclasses.replace(cp, needs_layout_passes=False)

  @functools.partial(
      pl.kernel,
      out_type=jax.ShapeDtypeStruct((M, C), src.dtype),
      mesh=mesh,
      compiler_params=cp,
  )
  def kern(x_hbm, i_hbm, o_hbm):
    def body(i_vmem, o_vmem):
      pltpu.sync_copy(x_hbm.at[i_vmem.at[0]], o_vmem)

    pltpu.emit_pipeline(
        body,
        grid=(M // GW,),
        in_specs=[pl.BlockSpec((1, GW), index_map=lambda i: (0, i))],
        out_specs=[pl.BlockSpec((GW, C), index_map=lambda i: (i, 0))],
        core_axis_name=("core", "subcore"),
        dimension_semantics=(pltpu.PARALLEL,),
    )(i_hbm, o_hbm)

  return kern(src, idx)


def _lrelu(h):
  return jnp.where(h >= 0, h, 0.2 * h)


def _gn_scale_shift(s1, s2, P, gamma, beta, cnt):
  mean = _dot(s1, P) / cnt
  e2 = _dot(s2, P) / cnt
  var = e2 - mean * mean
  inv = lax.rsqrt(var + _EPS)
  sc = gamma * inv
  return sc, beta - mean * sc


def _edge_kernel(two, xg_hbm, xt_ref, WeT_ref, ga_ref, ba_ref,
                 P_ref, *rest, K, N, Cg):
  if two:
    W2T_ref, g2_ref, b2_ref, out_ref, zs, buf, sem = rest
  else:
    out_ref, zs, buf, sem = rest
  b = pl.program_id(0)
  xc = xt_ref[0]
  WeT = WeT_ref[...]
  z0 = jnp.zeros((1, 128), jnp.float32)
  JJ = K // 2
  cnt = 2.0 * N * K

  def p1(jj, carry):
    s1, s2 = carry
    cp = pltpu.make_async_copy(xg_hbm.at[b, pl.ds(2 * jj, 2)], buf, sem)
    cp.start()
    cp.wait()
    ea = jnp.concatenate([buf[0, :, :Cg] - xc, xc], axis=1)
    eb = jnp.concatenate([buf[1, :, :Cg] - xc, xc], axis=1)
    za = _dotb(ea, WeT)
    zb = _dotb(eb, WeT)
    z = jnp.concatenate([za, zb], axis=1)
    zs[pl.ds(jj * N, N), :] = z
    return s1 + jnp.sum(z, 0, keepdims=True), s2 + jnp.sum(z * z, 0,
                                                           keepdims=True)

  s1, s2 = lax.fori_loop(0, JJ, p1, (z0, z0))
  sc, sh = _gn_scale_shift(s1, s2, P_ref[...], ga_ref[...], ba_ref[...], cnt)

  if two:
    W2T = W2T_ref[...]

    def p2(jj, carry):
      s1, s2 = carry
      h = _lrelu(zs[pl.ds(jj * N, N), :] * sc + sh)
      ya = _dotb(h[:, :64], W2T)
      yb = _dotb(h[:, 64:], W2T)
      y = jnp.concatenate([ya, yb], axis=1)
      zs[pl.ds(jj * N, N), :] = y
      return s1 + jnp.sum(y, 0, keepdims=True), s2 + jnp.sum(y * y, 0,
                                                             keepdims=True)

    s1, s2 = lax.fori_loop(0, JJ, p2, (z0, z0))
    sc, sh = _gn_scale_shift(s1, s2, P_ref[...], g2_ref[...], b2_ref[...],
                             cnt)

  def p3(jj, acc):
    h = _lrelu(zs[pl.ds(jj * N, N), :] * sc + sh)
    return jnp.maximum(acc, jnp.maximum(h[:, :64], h[:, 64:]))

  out_ref[0] = lax.fori_loop(0, JJ, p3,
                             jnp.full((N, 64), -jnp.inf, jnp.float32))


def _pairmat2(groups):
  g = np.arange(128) % 64 // (64 // groups)
  return jnp.asarray((g[:, None] == g[None, :]).astype(np.float32))


def _dup(v):
  return jnp.concatenate([v, v]).reshape(1, 128)


def _edge_block(xg, xt, WeT, ga, ba, W2T=None, g2=None, b2=None):
  B, K, N, Cw = xg.shape
  Cg = xt.shape[2]
  two = W2T is not None
  P = _pairmat2(32)
  ops = [xg, xt, WeT, _dup(ga), _dup(ba), P]
  in_specs = [
      pl.BlockSpec(memory_space=pl.ANY),
      pl.BlockSpec((1, N, Cg), lambda b: (b, 0, 0)),
      pl.BlockSpec(WeT.shape, lambda b: (0, 0)),
      pl.BlockSpec((1, 128), lambda b: (0, 0)),
      pl.BlockSpec((1, 128), lambda b: (0, 0)),
      pl.BlockSpec(P.shape, lambda b: (0, 0)),
  ]
  if two:
    ops += [W2T, _dup(g2), _dup(b2)]
    in_specs += [
        pl.BlockSpec(W2T.shape, lambda b: (0, 0)),
        pl.BlockSpec((1, 128), lambda b: (0, 0)),
        pl.BlockSpec((1, 128), lambda b: (0, 0)),
    ]
  return pl.pallas_call(
      functools.partial(_edge_kernel, two, K=K, N=N, Cg=Cg),
      grid=(B,),
      in_specs=in_specs,
      out_specs=pl.BlockSpec((1, N, 64), lambda b: (b, 0, 0)),
      out_shape=jax.ShapeDtypeStruct((B, N, 64), jnp.float32),
      scratch_shapes=[
          pltpu.VMEM((K // 2 * N, 128), jnp.float32),
          pltpu.VMEM((2, N, Cw), jnp.float32),
          pltpu.SemaphoreType.DMA,
      ],
      compiler_params=pltpu.CompilerParams(
          dimension_semantics=("parallel",)),
  )(*ops)


def _head6_kernel(xc_ref, W6T_ref, g6_ref, b6_ref, P6_ref, out_ref, zs,
                  *, N, TN):
  nt = N // TN
  z0 = jnp.zeros((1, 1024), jnp.float32)
  W6T = W6T_ref[...]

  def p1(t, carry):
    s1, s2 = carry
    y = _dotb(xc_ref[0, pl.ds(t * TN, TN), :], W6T)
    zs[pl.ds(t * TN, TN), :] = y
    return s1 + jnp.sum(y, 0, keepdims=True), s2 + jnp.sum(y * y, 0,
                                                           keepdims=True)

  s1, s2 = lax.fori_loop(0, nt, p1, (z0, z0))
  sc, sh = _gn_scale_shift(s1, s2, P6_ref[...], g6_ref[...], b6_ref[...],
                           32.0 * N)

  def p2(t, acc):
    h = _lrelu(zs[pl.ds(t * TN, TN), :] * sc + sh)
    return jnp.maximum(acc, jnp.max(h, axis=0, keepdims=True))

  out_ref[0] = lax.fori_loop(0, nt, p2,
                             jnp.full((1, 1024), -jnp.inf, jnp.float32))


def _head6(xcat, W6T, g6, b6):
  B, N, _ = xcat.shape
  P6 = _pairmat(1024, 32)
  return pl.pallas_call(
      functools.partial(_head6_kernel, N=N, TN=512),
      grid=(B,),
      in_specs=[
          pl.BlockSpec((1, N, xcat.shape[2]), lambda b: (b, 0, 0)),
          pl.BlockSpec(W6T.shape, lambda b: (0, 0)),
          pl.BlockSpec((1, 1024), lambda b: (0, 0)),
          pl.BlockSpec((1, 1024), lambda b: (0, 0)),
          pl.BlockSpec(P6.shape, lambda b: (0, 0)),
      ],
      out_specs=pl.BlockSpec((1, 1, 1024), lambda b: (b, 0, 0)),
      out_shape=jax.ShapeDtypeStruct((B, 1, 1024), jnp.float32),
      scratch_shapes=[pltpu.VMEM((N, 1024), jnp.float32)],
      compiler_params=pltpu.CompilerParams(
          dimension_semantics=("parallel",)),
  )(xcat, W6T, _row(g6), _row(b6), P6)


def _head7_kernel(xc_ref, fm_ref, WaT_ref, WbT_ref, g7_ref, b7_ref, P7_ref,
                  out_ref, zs, *, N, TN):
  nt = N // TN
  z0 = jnp.zeros((1, 512), jnp.float32)
  c7 = _dotb(fm_ref[0], WaT_ref[...])
  WbT = WbT_ref[...]

  def p1(t, carry):
    s1, s2 = carry
    y = _dotb(xc_ref[0, pl.ds(t * TN, TN), :], WbT) + c7
    zs[pl.ds(t * TN, TN), :] = y
    return s1 + jnp.sum(y, 0, keepdims=True), s2 + jnp.sum(y * y, 0,
                                                           keepdims=True)

  s1, s2 = lax.fori_loop(0, nt, p1, (z0, z0))
  sc, sh = _gn_scale_shift(s1, s2, P7_ref[...], g7_ref[...], b7_ref[...],
                           16.0 * N)

  def p2(t, _):
    out_ref[0, pl.ds(t * TN, TN), :] = _lrelu(
        zs[pl.ds(t * TN, TN), :] * sc + sh)
    return 0

  lax.fori_loop(0, nt, p2, 0)


def _head7(xcat, fmax, WaT, WbT, g7, b7):
  B, N, _ = xcat.shape
  P7 = _pairmat(512, 32)
  return pl.pallas_call(
      functools.partial(_head7_kernel, N=N, TN=512),
      grid=(B,),
      in_specs=[
          pl.BlockSpec((1, N, xcat.shape[2]), lambda b: (b, 0, 0)),
          pl.BlockSpec((1, 1, 1024), lambda b: (b, 0, 0)),
          pl.BlockSpec(WaT.shape, lambda b: (0, 0)),
          pl.BlockSpec(WbT.shape, lambda b: (0, 0)),
          pl.BlockSpec((1, 512), lambda b: (0, 0)),
          pl.BlockSpec((1, 512), lambda b: (0, 0)),
          pl.BlockSpec(P7.shape, lambda b: (0, 0)),
      ],
      out_specs=pl.BlockSpec((1, N, 512), lambda b: (b, 0, 0)),
      out_shape=jax.ShapeDtypeStruct((B, N, 512), jnp.float32),
      scratch_shapes=[pltpu.VMEM((N, 512), jnp.float32)],
      compiler_params=pltpu.CompilerParams(
          dimension_semantics=("parallel",)),
  )(xcat, fmax, WaT, WbT, _row(g7), _row(b7), P7)


def _flat_idx(idx, B, N):
  base = (jnp.arange(B, dtype=jnp.int32) * N)[:, None, None]
  return (jnp.transpose(idx, (0, 2, 1)) + base).reshape(1, -1)


@jax.jit
def _run(x, W1, g1, b1, W2, g2, b2, W3, g3, b3, W4, g4, b4, W5, g5, b5,
         W6, g6, b6, W7, g7, b7):
  B, _, N = x.shape
  xt6 = jnp.transpose(x, (0, 2, 1))
  xt3p = jnp.pad(xt6[:, :, :3], ((0, 0), (0, 0), (0, 5)))
  xt6p = jnp.pad(xt6, ((0, 0), (0, 0), (0, 10)))

  idx1 = _knn_topk(xt3p, _K)
  src1 = jnp.pad(xt6p, ((0, 0), (0, 0), (0, 112))).reshape(B * N, 128)
  xg1 = _sc_gather(src1, _flat_idx(idx1, B, N))
  WeT1 = jnp.concatenate([jnp.pad(W1[:, :6].T, ((0, 10), (0, 0))),
                          jnp.pad(W1[:, 6:].T, ((0, 10), (0, 0)))], axis=0)
  x1 = _edge_block(xg1.reshape(B, _K, N, 128), xt6p, WeT1, g1, b1,
                   W2.T, g2, b2)

  idx2 = _knn_topk(x1, _K)
  src2 = jnp.pad(x1, ((0, 0), (0, 0), (0, 64))).reshape(B * N, 128)
  xg2 = _sc_gather(src2, _flat_idx(idx2, B, N))
  x2 = _edge_block(xg2.reshape(B, _K, N, 128), x1, W3.T, g3, b3,
                   W4.T, g4, b4)

  idx3 = _knn_topk(x2, _K)
  src3 = jnp.pad(x2, ((0, 0), (0, 0), (0, 64))).reshape(B * N, 128)
  xg3 = _sc_gather(src3, _flat_idx(idx3, B, N))
  x3 = _edge_block(xg3.reshape(B, _K, N, 128), x2, W5.T, g5, b5)

  xcat = jnp.concatenate([x1, x2, x3], axis=2)
  fmax = _head6(xcat, W6.T, g6, b6)
  out = _head7(xcat, fmax, W7[:, :1024].T, W7[:, 1024:].T, g7, b7)
  return jnp.transpose(out, (0, 2, 1))


def kernel(x, W1, g1, b1, W2, g2, b2, W3, g3, b3, W4, g4, b4, W5, g5, b5,
           W6, g6, b6, W7, g7, b7, device=0):
  return _run(x, W1, g1, b1, W2, g2, b2, W3, g3, b3, W4, g4, b4,
              W5, g5, b5, W6, g6, b6, W7, g7, b7)


# speedup vs baseline: 11.2593x; 1.1674x over previous
import dataclasses
import functools

import jax
import jax.numpy as jnp
import numpy as np
from jax import lax
from jax.experimental import pallas as pl
from jax.experimental.pallas import tpu as pltpu
from jax.experimental.pallas import tpu_sc as plsc

_K = 20
_EPS = 1e-5


def _pairmat(C, groups):
  g = np.arange(C) // (C // groups)
  return jnp.asarray((g[:, None] == g[None, :]).astype(np.float32))


def _row(v):
  return v.reshape(1, -1)


def _dot(a, b):
  return jnp.dot(a, b, preferred_element_type=jnp.float32,
                 precision=lax.Precision.HIGHEST)


_BF = jnp.bfloat16


def _dotb(a, b):
  return jnp.dot(a.astype(_BF), b.astype(_BF),
                 preferred_element_type=jnp.float32)


def _knn_kernel(xt_ref, idx_ref, *, TR, k, N, C):
  r = pl.program_id(1)
  xall = xt_ref[0]
  rows = xt_ref[0, pl.ds(r * TR, TR), :]
  d = lax.dot_general(rows.astype(_BF), xall.astype(_BF),
                      (((1,), (1,)), ((), ())),
                      preferred_element_type=jnp.float32)
  ones = jnp.ones((1, C), jnp.float32)
  xx = lax.dot_general(ones, xall * xall, (((1,), (1,)), ((), ())),
                       preferred_element_type=jnp.float32,
                       precision=lax.Precision.HIGHEST)
  xxc = jnp.sum(rows * rows, axis=1, keepdims=True)
  s = (2.0 * d - xx) - xxc
  colid = lax.broadcasted_iota(jnp.int32, (TR, N), 1)
  cols = []
  for t in range(k):
    m = jnp.max(s, axis=1, keepdims=True)
    hit = s >= m
    a = jnp.min(jnp.where(hit, colid, N), axis=1, keepdims=True)
    cols.append(a)
    if t < k - 1:
      s = jnp.where(hit, -jnp.inf, s)
  idx_ref[0] = jnp.concatenate(cols, axis=1)


def _knn_topk(xt, k):
  B, N, C = xt.shape
  TR = 256
  return pl.pallas_call(
      functools.partial(_knn_kernel, TR=TR, k=k, N=N, C=C),
      grid=(B, N // TR),
      in_specs=[pl.BlockSpec((1, N, C), lambda b, r: (b, 0, 0))],
      out_specs=pl.BlockSpec((1, TR, k), lambda b, r: (b, r, 0)),
      out_shape=jax.ShapeDtypeStruct((B, N, k), jnp.int32),
      compiler_params=pltpu.CompilerParams(
          dimension_semantics=("parallel", "arbitrary")),
  )(xt)


def _sc_gather(src, idx):
  M = idx.shape[1]
  C = src.shape[1]
  GW = 128
  mesh = plsc.VectorSubcoreMesh(core_axis_name="core",
                                subcore_axis_name="subcore")
  cp = pltpu.CompilerParams()
  if "needs_layout_passes" in type(cp).__dataclass_fields__:
    cp = dataclasses.replace(cp, needs_layout_passes=False)

  @functools.partial(
      pl.kernel,
      out_type=jax.ShapeDtypeStruct((M, C), src.dtype),
      mesh=mesh,
      compiler_params=cp,
  )
  def kern(x_hbm, i_hbm, o_hbm):
    def body(i_vmem, o_vmem):
      pltpu.sync_copy(x_hbm.at[i_vmem.at[0]], o_vmem)

    pltpu.emit_pipeline(
        body,
        grid=(M // GW,),
        in_specs=[pl.BlockSpec((1, GW), index_map=lambda i: (0, i))],
        out_specs=[pl.BlockSpec((GW, C), index_map=lambda i: (i, 0))],
        core_axis_name=("core", "subcore"),
        dimension_semantics=(pltpu.PARALLEL,),
    )(i_hbm, o_hbm)

  return kern(src, idx)


def _lrelu(h):
  return jnp.where(h >= 0, h, 0.2 * h)


def _gn_scale_shift(s1, s2, P, gamma, beta, cnt):
  mean = _dot(s1, P) / cnt
  e2 = _dot(s2, P) / cnt
  var = e2 - mean * mean
  inv = lax.rsqrt(var + _EPS)
  sc = gamma * inv
  return sc, beta - mean * sc


def _edge_kernel(two, xg_hbm, xt_ref, WeT_ref, ga_ref, ba_ref,
                 P_ref, *rest, K, N, Cg):
  if two:
    W2T_ref, g2_ref, b2_ref, out_ref, zs, buf, sem = rest
  else:
    out_ref, zs, buf, sem = rest
  b = pl.program_id(0)
  xc = xt_ref[0]
  WeT = WeT_ref[...]
  z0 = jnp.zeros((1, 128), jnp.float32)
  JJ = K // 2
  cnt = 2.0 * N * K

  pltpu.make_async_copy(xg_hbm.at[b, pl.ds(0, 2)], buf.at[0],
                        sem.at[0]).start()

  def p1(jj, carry):
    s1, s2 = carry

    @pl.when(jj + 1 < JJ)
    def _():
      pltpu.make_async_copy(xg_hbm.at[b, pl.ds(2 * (jj + 1), 2)],
                            buf.at[(jj + 1) % 2],
                            sem.at[(jj + 1) % 2]).start()

    pltpu.make_async_copy(xg_hbm.at[b, pl.ds(2 * jj, 2)],
                          buf.at[jj % 2], sem.at[jj % 2]).wait()
    ea = jnp.concatenate([buf[jj % 2, 0, :, :Cg] - xc, xc], axis=1)
    eb = jnp.concatenate([buf[jj % 2, 1, :, :Cg] - xc, xc], axis=1)
    za = _dotb(ea, WeT)
    zb = _dotb(eb, WeT)
    z = jnp.concatenate([za, zb], axis=1)
    zs[pl.ds(jj * N, N), :] = z
    return s1 + jnp.sum(z, 0, keepdims=True), s2 + jnp.sum(z * z, 0,
                                                           keepdims=True)

  s1, s2 = lax.fori_loop(0, JJ, p1, (z0, z0))
  sc, sh = _gn_scale_shift(s1, s2, P_ref[...], ga_ref[...], ba_ref[...], cnt)

  if two:
    W2T = W2T_ref[...]

    def p2(jj, carry):
      s1, s2 = carry
      h = _lrelu(zs[pl.ds(jj * N, N), :] * sc + sh)
      ya = _dotb(h[:, :64], W2T)
      yb = _dotb(h[:, 64:], W2T)
      y = jnp.concatenate([ya, yb], axis=1)
      zs[pl.ds(jj * N, N), :] = y
      return s1 + jnp.sum(y, 0, keepdims=True), s2 + jnp.sum(y * y, 0,
                                                             keepdims=True)

    s1, s2 = lax.fori_loop(0, JJ, p2, (z0, z0))
    sc, sh = _gn_scale_shift(s1, s2, P_ref[...], g2_ref[...], b2_ref[...],
                             cnt)

  def p3(jj, acc):
    h = _lrelu(zs[pl.ds(jj * N, N), :] * sc + sh)
    return jnp.maximum(acc, jnp.maximum(h[:, :64], h[:, 64:]))

  out_ref[0] = lax.fori_loop(0, JJ, p3,
                             jnp.full((N, 64), -jnp.inf, jnp.float32))


def _pairmat2(groups):
  g = np.arange(128) % 64 // (64 // groups)
  return jnp.asarray((g[:, None] == g[None, :]).astype(np.float32))


def _dup(v):
  return jnp.concatenate([v, v]).reshape(1, 128)


def _edge_block(xg, xt, WeT, ga, ba, W2T=None, g2=None, b2=None):
  B, K, N, Cw = xg.shape
  Cg = xt.shape[2]
  two = W2T is not None
  P = _pairmat2(32)
  ops = [xg, xt, WeT, _dup(ga), _dup(ba), P]
  in_specs = [
      pl.BlockSpec(memory_space=pl.ANY),
      pl.BlockSpec((1, N, Cg), lambda b: (b, 0, 0)),
      pl.BlockSpec(WeT.shape, lambda b: (0, 0)),
      pl.BlockSpec((1, 128), lambda b: (0, 0)),
      pl.BlockSpec((1, 128), lambda b: (0, 0)),
      pl.BlockSpec(P.shape, lambda b: (0, 0)),
  ]
  if two:
    ops += [W2T, _dup(g2), _dup(b2)]
    in_specs += [
        pl.BlockSpec(W2T.shape, lambda b: (0, 0)),
        pl.BlockSpec((1, 128), lambda b: (0, 0)),
        pl.BlockSpec((1, 128), lambda b: (0, 0)),
    ]
  return pl.pallas_call(
      functools.partial(_edge_kernel, two, K=K, N=N, Cg=Cg),
      grid=(B,),
      in_specs=in_specs,
      out_specs=pl.BlockSpec((1, N, 64), lambda b: (b, 0, 0)),
      out_shape=jax.ShapeDtypeStruct((B, N, 64), jnp.float32),
      scratch_shapes=[
          pltpu.VMEM((K // 2 * N, 128), jnp.float32),
          pltpu.VMEM((2, 2, N, Cw), jnp.float32),
          pltpu.SemaphoreType.DMA((2,)),
      ],
      compiler_params=pltpu.CompilerParams(
          dimension_semantics=("parallel",)),
  )(*ops)


def _head6_kernel(xc_ref, W6T_ref, g6_ref, b6_ref, P6_ref, out_ref, zs,
                  *, N, TN):
  nt = N // TN
  z0 = jnp.zeros((1, 1024), jnp.float32)
  W6T = W6T_ref[...]

  def p1(t, carry):
    s1, s2 = carry
    y = _dotb(xc_ref[0, pl.ds(t * TN, TN), :], W6T)
    zs[pl.ds(t * TN, TN), :] = y
    return s1 + jnp.sum(y, 0, keepdims=True), s2 + jnp.sum(y * y, 0,
                                                           keepdims=True)

  s1, s2 = lax.fori_loop(0, nt, p1, (z0, z0))
  sc, sh = _gn_scale_shift(s1, s2, P6_ref[...], g6_ref[...], b6_ref[...],
                           32.0 * N)

  def p2(t, acc):
    h = _lrelu(zs[pl.ds(t * TN, TN), :] * sc + sh)
    return jnp.maximum(acc, jnp.max(h, axis=0, keepdims=True))

  out_ref[0] = lax.fori_loop(0, nt, p2,
                             jnp.full((1, 1024), -jnp.inf, jnp.float32))


def _head6(xcat, W6T, g6, b6):
  B, N, _ = xcat.shape
  P6 = _pairmat(1024, 32)
  return pl.pallas_call(
      functools.partial(_head6_kernel, N=N, TN=512),
      grid=(B,),
      in_specs=[
          pl.BlockSpec((1, N, xcat.shape[2]), lambda b: (b, 0, 0)),
          pl.BlockSpec(W6T.shape, lambda b: (0, 0)),
          pl.BlockSpec((1, 1024), lambda b: (0, 0)),
          pl.BlockSpec((1, 1024), lambda b: (0, 0)),
          pl.BlockSpec(P6.shape, lambda b: (0, 0)),
      ],
      out_specs=pl.BlockSpec((1, 1, 1024), lambda b: (b, 0, 0)),
      out_shape=jax.ShapeDtypeStruct((B, 1, 1024), jnp.float32),
      scratch_shapes=[pltpu.VMEM((N, 1024), jnp.float32)],
      compiler_params=pltpu.CompilerParams(
          dimension_semantics=("parallel",)),
  )(xcat, W6T, _row(g6), _row(b6), P6)


def _head7_kernel(xc_ref, fm_ref, WaT_ref, WbT_ref, g7_ref, b7_ref, P7_ref,
                  out_ref, zs, *, N, TN):
  nt = N // TN
  z0 = jnp.zeros((1, 512), jnp.float32)
  c7 = _dotb(fm_ref[0], WaT_ref[...])
  WbT = WbT_ref[...]

  def p1(t, carry):
    s1, s2 = carry
    y = _dotb(xc_ref[0, pl.ds(t * TN, TN), :], WbT) + c7
    zs[pl.ds(t * TN, TN), :] = y
    return s1 + jnp.sum(y, 0, keepdims=True), s2 + jnp.sum(y * y, 0,
                                                           keepdims=True)

  s1, s2 = lax.fori_loop(0, nt, p1, (z0, z0))
  sc, sh = _gn_scale_shift(s1, s2, P7_ref[...], g7_ref[...], b7_ref[...],
                           16.0 * N)

  def p2(t, _):
    out_ref[0, pl.ds(t * TN, TN), :] = _lrelu(
        zs[pl.ds(t * TN, TN), :] * sc + sh)
    return 0

  lax.fori_loop(0, nt, p2, 0)


def _head7(xcat, fmax, WaT, WbT, g7, b7):
  B, N, _ = xcat.shape
  P7 = _pairmat(512, 32)
  return pl.pallas_call(
      functools.partial(_head7_kernel, N=N, TN=512),
      grid=(B,),
      in_specs=[
          pl.BlockSpec((1, N, xcat.shape[2]), lambda b: (b, 0, 0)),
          pl.BlockSpec((1, 1, 1024), lambda b: (b, 0, 0)),
          pl.BlockSpec(WaT.shape, lambda b: (0, 0)),
          pl.BlockSpec(WbT.shape, lambda b: (0, 0)),
          pl.BlockSpec((1, 512), lambda b: (0, 0)),
          pl.BlockSpec((1, 512), lambda b: (0, 0)),
          pl.BlockSpec(P7.shape, lambda b: (0, 0)),
      ],
      out_specs=pl.BlockSpec((1, N, 512), lambda b: (b, 0, 0)),
      out_shape=jax.ShapeDtypeStruct((B, N, 512), jnp.float32),
      scratch_shapes=[pltpu.VMEM((N, 512), jnp.float32)],
      compiler_params=pltpu.CompilerParams(
          dimension_semantics=("parallel",)),
  )(xcat, fmax, WaT, WbT, _row(g7), _row(b7), P7)


def _flat_idx(idx, B, N):
  base = (jnp.arange(B, dtype=jnp.int32) * N)[:, None, None]
  return (jnp.transpose(idx, (0, 2, 1)) + base).reshape(1, -1)


def _pipeline(x, W1, g1, b1, W2, g2, b2, W3, g3, b3, W4, g4, b4, W5, g5, b5,
              W6, g6, b6, W7, g7, b7):
  B, _, N = x.shape
  xt6 = jnp.transpose(x, (0, 2, 1))
  xt3p = jnp.pad(xt6[:, :, :3], ((0, 0), (0, 0), (0, 5)))
  xt6p = jnp.pad(xt6, ((0, 0), (0, 0), (0, 10)))

  idx1 = _knn_topk(xt3p, _K)
  src1 = jnp.pad(xt6p, ((0, 0), (0, 0), (0, 112))).reshape(B * N, 128)
  xg1 = _sc_gather(src1, _flat_idx(idx1, B, N))
  WeT1 = jnp.concatenate([jnp.pad(W1[:, :6].T, ((0, 10), (0, 0))),
                          jnp.pad(W1[:, 6:].T, ((0, 10), (0, 0)))], axis=0)
  x1 = _edge_block(xg1.reshape(B, _K, N, 128), xt6p, WeT1, g1, b1,
                   W2.T, g2, b2)

  idx2 = _knn_topk(x1, _K)
  src2 = jnp.pad(x1, ((0, 0), (0, 0), (0, 64))).reshape(B * N, 128)
  xg2 = _sc_gather(src2, _flat_idx(idx2, B, N))
  x2 = _edge_block(xg2.reshape(B, _K, N, 128), x1, W3.T, g3, b3,
                   W4.T, g4, b4)

  idx3 = _knn_topk(x2, _K)
  src3 = jnp.pad(x2, ((0, 0), (0, 0), (0, 64))).reshape(B * N, 128)
  xg3 = _sc_gather(src3, _flat_idx(idx3, B, N))
  x3 = _edge_block(xg3.reshape(B, _K, N, 128), x2, W5.T, g5, b5)

  xcat = jnp.concatenate([x1, x2, x3], axis=2)
  fmax = _head6(xcat, W6.T, g6, b6)
  out = _head7(xcat, fmax, W7[:, :1024].T, W7[:, 1024:].T, g7, b7)
  return jnp.transpose(out, (0, 2, 1))


@jax.jit
def _run(x, *w):
  B = x.shape[0]
  h = B // 2
  return jnp.concatenate([_pipeline(x[:h], *w), _pipeline(x[h:], *w)],
                         axis=0)


def kernel(x, W1, g1, b1, W2, g2, b2, W3, g3, b3, W4, g4, b4, W5, g5, b5,
           W6, g6, b6, W7, g7, b7, device=0):
  return _run(x, W1, g1, b1, W2, g2, b2, W3, g3, b3, W4, g4, b4,
              W5, g5, b5, W6, g6, b6, W7, g7, b7)


# speedup vs baseline: 11.4798x; 1.0196x over previous
import dataclasses
import functools

import jax
import jax.numpy as jnp
import numpy as np
from jax import lax
from jax.experimental import pallas as pl
from jax.experimental.pallas import tpu as pltpu
from jax.experimental.pallas import tpu_sc as plsc

_K = 20
_EPS = 1e-5


def _pairmat(C, groups):
  g = np.arange(C) // (C // groups)
  return jnp.asarray((g[:, None] == g[None, :]).astype(np.float32))


def _row(v):
  return v.reshape(1, -1)


def _dot(a, b):
  return jnp.dot(a, b, preferred_element_type=jnp.float32,
                 precision=lax.Precision.HIGHEST)


_BF = jnp.bfloat16


def _dotb(a, b):
  return jnp.dot(a.astype(_BF), b.astype(_BF),
                 preferred_element_type=jnp.float32)


def _knn_kernel(xc_ref, xt_ref, idx_ref, *, TR, k, N, C):
  r = pl.program_id(1)
  xall = xc_ref[0]
  ctile = xc_ref[0, :, pl.ds(r * TR, TR)]
  d = lax.dot_general(ctile.astype(_BF), xall.astype(_BF),
                      (((0,), (0,)), ((), ())),
                      preferred_element_type=jnp.float32)
  xx = jnp.sum(xall * xall, axis=0, keepdims=True)
  rows = xt_ref[0, pl.ds(r * TR, TR), :]
  xxc = jnp.sum(rows * rows, axis=1, keepdims=True)
  s = (2.0 * d - xx) - xxc
  colid = lax.broadcasted_iota(jnp.int32, (TR, N), 1)
  cols = []
  for t in range(k):
    m = jnp.max(s, axis=1, keepdims=True)
    hit = s >= m
    a = jnp.min(jnp.where(hit, colid, N), axis=1, keepdims=True)
    cols.append(a)
    if t < k - 1:
      s = jnp.where(hit, -jnp.inf, s)
  idx_ref[0] = jnp.concatenate(cols, axis=1)


def _knn_topk(xc, xt, k):
  B, C, N = xc.shape
  TR = 256
  return pl.pallas_call(
      functools.partial(_knn_kernel, TR=TR, k=k, N=N, C=C),
      grid=(B, N // TR),
      in_specs=[pl.BlockSpec((1, C, N), lambda b, r: (b, 0, 0)),
                pl.BlockSpec((1, N, C), lambda b, r: (b, 0, 0))],
      out_specs=pl.BlockSpec((1, TR, k), lambda b, r: (b, r, 0)),
      out_shape=jax.ShapeDtypeStruct((B, N, k), jnp.int32),
      compiler_params=pltpu.CompilerParams(
          dimension_semantics=("parallel", "arbitrary")),
  )(xc, xt)


def _sc_gather(src, idx):
  M = idx.shape[1]
  C = src.shape[1]
  GW = 128
  mesh = plsc.VectorSubcoreMesh(core_axis_name="core",
                                subcore_axis_name="subcore")
  cp = pltpu.CompilerParams()
  if "needs_layout_passes" in type(cp).__dataclass_fields__:
    cp = dataclasses.replace(cp, needs_layout_passes=False)

  @functools.partial(
      pl.kernel,
      out_type=jax.ShapeDtypeStruct((M, C), src.dtype),
      mesh=mesh,
      compiler_params=cp,
  )
  def kern(x_hbm, i_hbm, o_hbm):
    def body(i_vmem, o_vmem):
      pltpu.sync_copy(x_hbm.at[i_vmem.at[0]], o_vmem)

    pltpu.emit_pipeline(
        body,
        grid=(M // GW,),
        in_specs=[pl.BlockSpec((1, GW), index_map=lambda i: (0, i))],
        out_specs=[pl.BlockSpec((GW, C), index_map=lambda i: (i, 0))],
        core_axis_name=("core", "subcore"),
        dimension_semantics=(pltpu.PARALLEL,),
    )(i_hbm, o_hbm)

  return kern(src, idx)


def _lrelu(h):
  return jnp.where(h >= 0, h, 0.2 * h)


def _gn_apply(z, mean, sq, gamma, beta):
  return ((z - mean) / sq) * gamma + beta


def _edge_kernel(two, xg_hbm, xt_ref, WeT_ref, ga_ref, ba_ref,
                 P_ref, *rest, K, N, Cg):
  if two:
    W2T_ref, g2_ref, b2_ref, out_ref, zs, buf, sem = rest
  else:
    out_ref, zs, buf, sem = rest
  b = pl.program_id(0)
  xc = xt_ref[0]
  WeT = WeT_ref[...]
  z0 = jnp.zeros((1, 128), jnp.float32)
  JJ = K // 2
  cnt = 2.0 * N * K

  pltpu.make_async_copy(xg_hbm.at[b, pl.ds(0, 2)], buf.at[0],
                        sem.at[0]).start()

  def _stats(P):
    def pm(jj, s1):
      return s1 + jnp.sum(zs[pl.ds(jj * N, N), :], 0, keepdims=True)

    mean = _dot(lax.fori_loop(0, JJ, pm, z0), P) / cnt

    def pv(jj, v):
      zc = zs[pl.ds(jj * N, N), :] - mean
      return v + jnp.sum(zc * zc, 0, keepdims=True)

    var = _dot(lax.fori_loop(0, JJ, pv, z0), P) / cnt
    return mean, jnp.sqrt(var + _EPS)

  def p1(jj, _):
    @pl.when(jj + 1 < JJ)
    def _start():
      pltpu.make_async_copy(xg_hbm.at[b, pl.ds(2 * (jj + 1), 2)],
                            buf.at[(jj + 1) % 2],
                            sem.at[(jj + 1) % 2]).start()

    pltpu.make_async_copy(xg_hbm.at[b, pl.ds(2 * jj, 2)],
                          buf.at[jj % 2], sem.at[jj % 2]).wait()
    ea = jnp.concatenate([buf[jj % 2, 0, :, :Cg] - xc, xc], axis=1)
    eb = jnp.concatenate([buf[jj % 2, 1, :, :Cg] - xc, xc], axis=1)
    za = _dotb(ea, WeT)
    zb = _dotb(eb, WeT)
    zs[pl.ds(jj * N, N), :] = jnp.concatenate([za, zb], axis=1)
    return 0

  lax.fori_loop(0, JJ, p1, 0)
  mean, inv = _stats(P_ref[...])
  ga, ba = ga_ref[...], ba_ref[...]

  if two:
    W2T = W2T_ref[...]

    def p2(jj, _):
      h = _lrelu(_gn_apply(zs[pl.ds(jj * N, N), :], mean, inv, ga, ba))
      ya = _dotb(h[:, :64], W2T)
      yb = _dotb(h[:, 64:], W2T)
      zs[pl.ds(jj * N, N), :] = jnp.concatenate([ya, yb], axis=1)
      return 0

    lax.fori_loop(0, JJ, p2, 0)
    mean, inv = _stats(P_ref[...])
    ga, ba = g2_ref[...], b2_ref[...]

  def p3(jj, acc):
    h = _lrelu(_gn_apply(zs[pl.ds(jj * N, N), :], mean, inv, ga, ba))
    return jnp.maximum(acc, jnp.maximum(h[:, :64], h[:, 64:]))

  out_ref[0] = lax.fori_loop(0, JJ, p3,
                             jnp.full((N, 64), -jnp.inf, jnp.float32))


def _pairmat2(groups):
  g = np.arange(128) % 64 // (64 // groups)
  return jnp.asarray((g[:, None] == g[None, :]).astype(np.float32))


def _dup(v):
  return jnp.concatenate([v, v]).reshape(1, 128)


def _edge_block(xg, xt, WeT, ga, ba, W2T=None, g2=None, b2=None):
  B, K, N, Cw = xg.shape
  Cg = xt.shape[2]
  two = W2T is not None
  P = _pairmat2(32)
  ops = [xg, xt, WeT, _dup(ga), _dup(ba), P]
  in_specs = [
      pl.BlockSpec(memory_space=pl.ANY),
      pl.BlockSpec((1, N, Cg), lambda b: (b, 0, 0)),
      pl.BlockSpec(WeT.shape, lambda b: (0, 0)),
      pl.BlockSpec((1, 128), lambda b: (0, 0)),
      pl.BlockSpec((1, 128), lambda b: (0, 0)),
      pl.BlockSpec(P.shape, lambda b: (0, 0)),
  ]
  if two:
    ops += [W2T, _dup(g2), _dup(b2)]
    in_specs += [
        pl.BlockSpec(W2T.shape, lambda b: (0, 0)),
        pl.BlockSpec((1, 128), lambda b: (0, 0)),
        pl.BlockSpec((1, 128), lambda b: (0, 0)),
    ]
  return pl.pallas_call(
      functools.partial(_edge_kernel, two, K=K, N=N, Cg=Cg),
      grid=(B,),
      in_specs=in_specs,
      out_specs=pl.BlockSpec((1, N, 64), lambda b: (b, 0, 0)),
      out_shape=jax.ShapeDtypeStruct((B, N, 64), jnp.float32),
      scratch_shapes=[
          pltpu.VMEM((K // 2 * N, 128), jnp.float32),
          pltpu.VMEM((2, 2, N, Cw), jnp.float32),
          pltpu.SemaphoreType.DMA((2,)),
      ],
      compiler_params=pltpu.CompilerParams(
          dimension_semantics=("parallel",)),
  )(*ops)


def _head6_kernel(xc_ref, W6T_ref, g6_ref, b6_ref, P6_ref, out_ref, zs,
                  *, N, TN):
  nt = N // TN
  z0 = jnp.zeros((1, 1024), jnp.float32)
  W6T = W6T_ref[...]

  def p1(t, s1):
    y = _dotb(xc_ref[0, pl.ds(t * TN, TN), :], W6T)
    zs[pl.ds(t * TN, TN), :] = y
    return s1 + jnp.sum(y, 0, keepdims=True)

  mean = _dot(lax.fori_loop(0, nt, p1, z0), P6_ref[...]) / (32.0 * N)

  def pv(t, v):
    zc = zs[pl.ds(t * TN, TN), :] - mean
    return v + jnp.sum(zc * zc, 0, keepdims=True)

  var = _dot(lax.fori_loop(0, nt, pv, z0), P6_ref[...]) / (32.0 * N)
  inv = jnp.sqrt(var + _EPS)

  def p2(t, acc):
    h = _lrelu(_gn_apply(zs[pl.ds(t * TN, TN), :], mean, inv,
                         g6_ref[...], b6_ref[...]))
    return jnp.maximum(acc, jnp.max(h, axis=0, keepdims=True))

  out_ref[0] = lax.fori_loop(0, nt, p2,
                             jnp.full((1, 1024), -jnp.inf, jnp.float32))


def _head6(xcat, W6T, g6, b6):
  B, N, _ = xcat.shape
  P6 = _pairmat(1024, 32)
  return pl.pallas_call(
      functools.partial(_head6_kernel, N=N, TN=512),
      grid=(B,),
      in_specs=[
          pl.BlockSpec((1, N, xcat.shape[2]), lambda b: (b, 0, 0)),
          pl.BlockSpec(W6T.shape, lambda b: (0, 0)),
          pl.BlockSpec((1, 1024), lambda b: (0, 0)),
          pl.BlockSpec((1, 1024), lambda b: (0, 0)),
          pl.BlockSpec(P6.shape, lambda b: (0, 0)),
      ],
      out_specs=pl.BlockSpec((1, 1, 1024), lambda b: (b, 0, 0)),
      out_shape=jax.ShapeDtypeStruct((B, 1, 1024), jnp.float32),
      scratch_shapes=[pltpu.VMEM((N, 1024), jnp.float32)],
      compiler_params=pltpu.CompilerParams(
          dimension_semantics=("parallel",)),
  )(xcat, W6T, _row(g6), _row(b6), P6)


def _head7_kernel(xc_ref, fm_ref, WaT_ref, WbT_ref, g7_ref, b7_ref, P7_ref,
                  out_ref, zs, *, N, TN):
  nt = N // TN
  z0 = jnp.zeros((1, 512), jnp.float32)
  c7 = _dotb(fm_ref[0], WaT_ref[...])
  WbT = WbT_ref[...]

  def p1(t, s1):
    y = _dotb(xc_ref[0, pl.ds(t * TN, TN), :], WbT) + c7
    zs[pl.ds(t * TN, TN), :] = y
    return s1 + jnp.sum(y, 0, keepdims=True)

  mean = _dot(lax.fori_loop(0, nt, p1, z0), P7_ref[...]) / (16.0 * N)

  def pv(t, v):
    zc = zs[pl.ds(t * TN, TN), :] - mean
    return v + jnp.sum(zc * zc, 0, keepdims=True)

  var = _dot(lax.fori_loop(0, nt, pv, z0), P7_ref[...]) / (16.0 * N)
  inv = jnp.sqrt(var + _EPS)

  def p2(t, _):
    out_ref[0, pl.ds(t * TN, TN), :] = _lrelu(
        _gn_apply(zs[pl.ds(t * TN, TN), :], mean, inv, g7_ref[...],
                  b7_ref[...]))
    return 0

  lax.fori_loop(0, nt, p2, 0)


def _head7(xcat, fmax, WaT, WbT, g7, b7):
  B, N, _ = xcat.shape
  P7 = _pairmat(512, 32)
  return pl.pallas_call(
      functools.partial(_head7_kernel, N=N, TN=512),
      grid=(B,),
      in_specs=[
          pl.BlockSpec((1, N, xcat.shape[2]), lambda b: (b, 0, 0)),
          pl.BlockSpec((1, 1, 1024), lambda b: (b, 0, 0)),
          pl.BlockSpec(WaT.shape, lambda b: (0, 0)),
          pl.BlockSpec(WbT.shape, lambda b: (0, 0)),
          pl.BlockSpec((1, 512), lambda b: (0, 0)),
          pl.BlockSpec((1, 512), lambda b: (0, 0)),
          pl.BlockSpec(P7.shape, lambda b: (0, 0)),
      ],
      out_specs=pl.BlockSpec((1, N, 512), lambda b: (b, 0, 0)),
      out_shape=jax.ShapeDtypeStruct((B, N, 512), jnp.float32),
      scratch_shapes=[pltpu.VMEM((N, 512), jnp.float32)],
      compiler_params=pltpu.CompilerParams(
          dimension_semantics=("parallel",)),
  )(xcat, fmax, WaT, WbT, _row(g7), _row(b7), P7)


def _flat_idx(idx, B, N):
  base = (jnp.arange(B, dtype=jnp.int32) * N)[:, None, None]
  return (jnp.transpose(idx, (0, 2, 1)) + base).reshape(1, -1)


def _pipeline(x, W1, g1, b1, W2, g2, b2, W3, g3, b3, W4, g4, b4, W5, g5, b5,
              W6, g6, b6, W7, g7, b7):
  B, _, N = x.shape
  xt6 = jnp.transpose(x, (0, 2, 1))
  xt3p = jnp.pad(xt6[:, :, :3], ((0, 0), (0, 0), (0, 5)))
  xc3p = jnp.pad(x[:, :3, :], ((0, 0), (0, 5), (0, 0)))
  xt6p = jnp.pad(xt6, ((0, 0), (0, 0), (0, 10)))

  idx1 = _knn_topk(xc3p, xt3p, _K)
  src1 = jnp.pad(xt6p, ((0, 0), (0, 0), (0, 112))).reshape(B * N, 128)
  xg1 = _sc_gather(src1, _flat_idx(idx1, B, N))
  WeT1 = jnp.concatenate([jnp.pad(W1[:, :6].T, ((0, 10), (0, 0))),
                          jnp.pad(W1[:, 6:].T, ((0, 10), (0, 0)))], axis=0)
  x1 = _edge_block(xg1.reshape(B, _K, N, 128), xt6p, WeT1, g1, b1,
                   W2.T, g2, b2)

  idx2 = _knn_topk(jnp.transpose(x1, (0, 2, 1)), x1, _K)
  src2 = jnp.pad(x1, ((0, 0), (0, 0), (0, 64))).reshape(B * N, 128)
  xg2 = _sc_gather(src2, _flat_idx(idx2, B, N))
  x2 = _edge_block(xg2.reshape(B, _K, N, 128), x1, W3.T, g3, b3,
                   W4.T, g4, b4)

  idx3 = _knn_topk(jnp.transpose(x2, (0, 2, 1)), x2, _K)
  src3 = jnp.pad(x2, ((0, 0), (0, 0), (0, 64))).reshape(B * N, 128)
  xg3 = _sc_gather(src3, _flat_idx(idx3, B, N))
  x3 = _edge_block(xg3.reshape(B, _K, N, 128), x2, W5.T, g5, b5)

  xcat = jnp.concatenate([x1, x2, x3], axis=2)
  fmax = _head6(xcat, W6.T, g6, b6)
  out = _head7(xcat, fmax, W7[:, :1024].T, W7[:, 1024:].T, g7, b7)
  return jnp.transpose(out, (0, 2, 1))


@jax.jit
def _run(x, *w):
  B = x.shape[0]
  h = B // 2
  return jnp.concatenate([_pipeline(x[:h], *w), _pipeline(x[h:], *w)],
                         axis=0)


def kernel(x, W1, g1, b1, W2, g2, b2, W3, g3, b3, W4, g4, b4, W5, g5, b5,
           W6, g6, b6, W7, g7, b7, device=0):
  return _run(x, W1, g1, b1, W2, g2, b2, W3, g3, b3, W4, g4, b4,
              W5, g5, b5, W6, g6, b6, W7, g7, b7)
